# baseline (speedup 1.0000x reference)
import functools

import jax
import jax.numpy as jnp
from jax import lax
from jax.experimental import pallas as pl
from jax.experimental.pallas import tpu as pltpu
from jax.experimental.pallas import tpu_sc as plsc

B, L, C = 2, 2048, 768
H, K, HALF_S = 12, 32, 4
S = 2 * HALF_S + 1
D = C // H
MAX_FREQ, MIN_FREQ = 16.0, 1.0
MAX_RECEPTIVE = HALF_S * MAX_FREQ

T = 256
NBLK = L // T
HALO = 96
WC = T + 2 * HALO
SP = 16
HS = H * SP
NW = 32
TASKS = B * H * NBLK
NC = 2


def _silu(v):
    return v * jax.nn.sigmoid(v)


def _a_body(x_ref, Ww_ref, bw_ref, gw_ref, Wk_ref, bk_ref, gk_ref,
            We_ref, be_ref, ge_ref, wf_ref, wc_ref, sfi_ref):
    blk = pl.program_id(1)
    l0 = blk * T
    xb = x_ref[0]

    pw = jnp.dot(xb, Ww_ref[...], preferred_element_type=jnp.float32) + bw_ref[...][None, :]
    var_w = jnp.sum(pw * pw, axis=-1, keepdims=True) / (2 * H)
    wave = _silu(gw_ref[...][None, :] * (pw * jax.lax.rsqrt(var_w + 1e-6)))
    freq = jax.nn.sigmoid(wave[:, :H]) * (MAX_FREQ - MIN_FREQ) + MIN_FREQ
    phase = jnp.tanh(wave[:, H:2 * H]) * MAX_FREQ

    pk = jnp.dot(xb, Wk_ref[...], preferred_element_type=jnp.float32) + bk_ref[...][None, :]
    var_k = jnp.sum(pk * pk, axis=-1, keepdims=True) / (H * K)
    km = _silu(gk_ref[...][None, :] * (pk * jax.lax.rsqrt(var_k + 1e-6)))

    pe = jnp.dot(xb, We_ref[...], preferred_element_type=jnp.float32) + be_ref[...][None, :]
    ve = pe[:, 0:1]
    ve_n = ge_ref[...][0:1][None, :] * (ve * jax.lax.rsqrt(ve * ve + 1e-6))
    exponent = jax.nn.sigmoid(ve_n) * 3.5 + 0.5

    centers = (l0 + jax.lax.broadcasted_iota(jnp.int32, (T, 1), 0)).astype(jnp.float32)
    iota_k = jax.lax.broadcasted_iota(jnp.int32, (T, K), 1)

    wf_cols, wc_cols, sfi_cols = [], [], []
    for h in range(H):
        fh = freq[:, h:h + 1]
        ph = phase[:, h:h + 1]
        kmh = km[:, h * K:(h + 1) * K]
        for s in range(S):
            stride = float(s - HALF_S)
            rel = stride * fh
            sp = centers + rel + ph
            valid = ((sp >= 0) & (sp < L)).astype(jnp.float32)
            pos_c = jnp.clip(sp, 0.0, L - 1.001)
            sfloor = jnp.clip(jnp.floor(pos_c).astype(jnp.int32), 0, L - 1)
            frac = pos_c - sfloor.astype(jnp.float32)
            nd = jnp.abs(rel) / L
            pwr = jnp.exp(-exponent * jnp.log1p(nd))
            np_ = jnp.clip(jnp.abs(rel) / MAX_RECEPTIVE, 0.0, 1.0)
            idx_f = np_ * (K - 1)
            idxf = jnp.clip(idx_f.astype(jnp.int32), 0, K - 2)
            w_ce = idx_f - idxf.astype(jnp.float32)
            kf = jnp.sum(jnp.where(iota_k == idxf, kmh, 0.0), axis=-1, keepdims=True)
            kc = jnp.sum(jnp.where(iota_k == idxf + 1, kmh, 0.0), axis=-1, keepdims=True)
            ker = (kf * (1.0 - w_ce) + kc * w_ce) * pwr * valid
            wf_cols.append(ker * (1.0 - frac))
            wc_cols.append(ker * frac)
            sfi_cols.append((sfloor - (l0 - HALO)) * D)
        for _ in range(SP - S):
            wf_cols.append(jnp.zeros((T, 1), jnp.float32))
            wc_cols.append(jnp.zeros((T, 1), jnp.float32))
            sfi_cols.append(jnp.zeros((T, 1), jnp.int32))
    wf_ref[0] = jnp.concatenate(wf_cols, axis=1)
    wc_ref[0] = jnp.concatenate(wc_cols, axis=1)
    sfi_ref[0] = jnp.concatenate(sfi_cols, axis=1)


def _stage_a(x, Ww, bw, gw, Wk, bk, gk, We, be, ge):
    full = lambda shape: pl.BlockSpec(shape, lambda b, i: (0,) * len(shape))
    return pl.pallas_call(
        _a_body,
        grid=(B, NBLK),
        in_specs=[
            pl.BlockSpec((1, T, C), lambda b, i: (b, i, 0)),
            full((C, 128)), full((128,)), full((128,)),
            full((C, H * K)), full((H * K,)), full((H * K,)),
            full((C, 128)), full((128,)), full((128,)),
        ],
        out_specs=[
            pl.BlockSpec((1, T, HS), lambda b, i: (b, i, 0)),
            pl.BlockSpec((1, T, HS), lambda b, i: (b, i, 0)),
            pl.BlockSpec((1, T, HS), lambda b, i: (b, i, 0)),
        ],
        out_shape=[
            jax.ShapeDtypeStruct((B, L, HS), jnp.float32),
            jax.ShapeDtypeStruct((B, L, HS), jnp.float32),
            jax.ShapeDtypeStruct((B, L, HS), jnp.int32),
        ],
    )(x, Ww, bw, gw, Wk, bk, gk, We, be, ge)


def _b_body(xh_ref, wf_ref, wc_ref, sfi_ref, out_ref,
            xwin, wfv, wcv, sfiv, outv):
    wid = lax.axis_index("s") * NC + lax.axis_index("c")

    def task_body(t, carry):
        tid = wid * (TASKS // NW) + t
        b = tid // (H * NBLK)
        r = tid % (H * NBLK)
        h = r // NBLK
        blk = r % NBLK
        pltpu.sync_copy(xh_ref.at[b, h, pl.ds(blk * T * D, WC * D)], xwin)
        pltpu.sync_copy(wf_ref.at[b, pl.ds(h * SP, SP), pl.ds(blk * T, T)], wfv)
        pltpu.sync_copy(wc_ref.at[b, pl.ds(h * SP, SP), pl.ds(blk * T, T)], wcv)
        pltpu.sync_copy(sfi_ref.at[b, pl.ds(h * SP, SP), pl.ds(blk * T, T)], sfiv)

        def l_body(g, carry2):
            base = g * 16
            wfs = [wfv[j, pl.ds(base, 16)] for j in range(S)]
            wcs = [wcv[j, pl.ds(base, 16)] for j in range(S)]
            idxf = [sfiv[j, pl.ds(base, 16)] for j in range(S)]
            idxc = [ix + D for ix in idxf]

            def d_body(d, carry3):
                acc = jnp.zeros((16,), jnp.float32)
                for j in range(S):
                    vf = plsc.load_gather(xwin, [idxf[j] + d])
                    vc = plsc.load_gather(xwin, [idxc[j] + d])
                    acc = acc + wfs[j] * vf + wcs[j] * vc
                outv[pl.ds(d * T + base, 16)] = acc
                return carry3

            lax.fori_loop(0, D, d_body, 0)
            return carry2

        lax.fori_loop(0, T // 16, l_body, 0)
        pltpu.sync_copy(outv, out_ref.at[b, h, blk])
        return carry

    lax.fori_loop(0, TASKS // NW, task_body, 0)


def _stage_b(xh, wf_t, wc_t, sfi_t):
    mesh = plsc.VectorSubcoreMesh(core_axis_name="c", subcore_axis_name="s",
                                  num_cores=NC, num_subcores=NW // NC)
    return pl.kernel(
        _b_body,
        out_type=jax.ShapeDtypeStruct((B, H, NBLK, D * T), jnp.float32),
        mesh=mesh,
        compiler_params=pltpu.CompilerParams(needs_layout_passes=False),
        scratch_types=[
            pltpu.VMEM((WC * D,), jnp.float32),
            pltpu.VMEM((SP, T), jnp.float32),
            pltpu.VMEM((SP, T), jnp.float32),
            pltpu.VMEM((SP, T), jnp.int32),
            pltpu.VMEM((D * T,), jnp.float32),
        ],
    )(xh, wf_t, wc_t, sfi_t)


def _c_body(xt_ref, wo_ref, o_ref):
    h = pl.program_id(2)

    @pl.when(h == 0)
    def _():
        o_ref[...] = jnp.zeros_like(o_ref)

    o_ref[0] += lax.dot_general(
        xt_ref[0, 0, 0], wo_ref[...],
        (((0,), (0,)), ((), ())),
        preferred_element_type=jnp.float32)


def _stage_c(acc_t, WoT):
    return pl.pallas_call(
        _c_body,
        grid=(B, NBLK, H),
        in_specs=[
            pl.BlockSpec((1, 1, 1, D, T), lambda b, i, h: (b, h, i, 0, 0)),
            pl.BlockSpec((D, C), lambda b, i, h: (h, 0)),
        ],
        out_specs=pl.BlockSpec((1, T, C), lambda b, i, h: (b, i, 0)),
        out_shape=jax.ShapeDtypeStruct((B, L, C), jnp.float32),
    )(acc_t, WoT)


@jax.jit
def kernel(x, W_wave, b_wave, wave_gamma, W_kernel, b_kernel, kernel_gamma,
           W_exp, b_exp, exp_gamma, W_out):
    f32 = jnp.float32
    Ww = jnp.zeros((C, 128), f32).at[:, :2 * H].set(W_wave.T)
    bw = jnp.zeros((128,), f32).at[:2 * H].set(b_wave)
    gw = jnp.zeros((128,), f32).at[:2 * H].set(wave_gamma)
    We = jnp.zeros((C, 128), f32).at[:, :1].set(W_exp.T)
    be = jnp.zeros((128,), f32).at[:1].set(b_exp)
    ge = jnp.zeros((128,), f32).at[:1].set(exp_gamma)

    wf, wc, sfi = _stage_a(x, Ww, bw, gw, W_kernel.T, b_kernel, kernel_gamma,
                           We, be, ge)
    wf_t = wf.transpose(0, 2, 1)
    wc_t = wc.transpose(0, 2, 1)
    sfi_t = sfi.transpose(0, 2, 1)

    x_pad = jnp.pad(x, ((0, 0), (HALO, HALO), (0, 0)))
    xh = (x_pad.reshape(B, L + 2 * HALO, H, D)
          .transpose(0, 2, 1, 3)
          .reshape(B, H, (L + 2 * HALO) * D))

    acc_t = _stage_b(xh, wf_t, wc_t, sfi_t)
    return _stage_c(acc_t.reshape(B, H, NBLK, D, T), W_out.T)


# speedup vs baseline: 1.0301x; 1.0301x over previous
import functools

import jax
import jax.numpy as jnp
from jax import lax
from jax.experimental import pallas as pl
from jax.experimental.pallas import tpu as pltpu
from jax.experimental.pallas import tpu_sc as plsc

B, L, C = 2, 2048, 768
H, K, HALF_S = 12, 32, 4
S = 2 * HALF_S + 1
D = C // H
MAX_FREQ, MIN_FREQ = 16.0, 1.0
MAX_RECEPTIVE = HALF_S * MAX_FREQ

T = 256
NBLK = L // T
HALO = 96
WC = T + 2 * HALO
SP = 16
HS = H * SP
NW = 32
TASKS = B * H * NBLK
NC = 2


def _silu(v):
    return v * jax.nn.sigmoid(v)


def _a_body(x_ref, Ww_ref, bw_ref, gw_ref, Wk_ref, bk_ref, gk_ref,
            We_ref, be_ref, ge_ref, wf_ref, wc_ref, sfi_ref):
    blk = pl.program_id(1)
    l0 = blk * T
    xb = x_ref[0]

    pw = jnp.dot(xb, Ww_ref[...], preferred_element_type=jnp.float32) + bw_ref[...][None, :]
    var_w = jnp.sum(pw * pw, axis=-1, keepdims=True) / (2 * H)
    wave = _silu(gw_ref[...][None, :] * (pw * jax.lax.rsqrt(var_w + 1e-6)))
    freq = jax.nn.sigmoid(wave[:, :H]) * (MAX_FREQ - MIN_FREQ) + MIN_FREQ
    phase = jnp.tanh(wave[:, H:2 * H]) * MAX_FREQ

    pk = jnp.dot(xb, Wk_ref[...], preferred_element_type=jnp.float32) + bk_ref[...][None, :]
    var_k = jnp.sum(pk * pk, axis=-1, keepdims=True) / (H * K)
    km = _silu(gk_ref[...][None, :] * (pk * jax.lax.rsqrt(var_k + 1e-6)))

    pe = jnp.dot(xb, We_ref[...], preferred_element_type=jnp.float32) + be_ref[...][None, :]
    ve = pe[:, 0:1]
    ve_n = ge_ref[...][0:1][None, :] * (ve * jax.lax.rsqrt(ve * ve + 1e-6))
    exponent = jax.nn.sigmoid(ve_n) * 3.5 + 0.5

    centers = (l0 + jax.lax.broadcasted_iota(jnp.int32, (T, 1), 0)).astype(jnp.float32)
    iota_k = jax.lax.broadcasted_iota(jnp.int32, (T, K), 1)

    wf_cols, wc_cols, sfi_cols = [], [], []
    for h in range(H):
        fh = freq[:, h:h + 1]
        ph = phase[:, h:h + 1]
        kmh = km[:, h * K:(h + 1) * K]
        for s in range(S):
            stride = float(s - HALF_S)
            rel = stride * fh
            sp = centers + rel + ph
            valid = ((sp >= 0) & (sp < L)).astype(jnp.float32)
            pos_c = jnp.clip(sp, 0.0, L - 1.001)
            sfloor = jnp.clip(jnp.floor(pos_c).astype(jnp.int32), 0, L - 1)
            frac = pos_c - sfloor.astype(jnp.float32)
            nd = jnp.abs(rel) / L
            pwr = jnp.exp(-exponent * jnp.log1p(nd))
            np_ = jnp.clip(jnp.abs(rel) / MAX_RECEPTIVE, 0.0, 1.0)
            idx_f = np_ * (K - 1)
            idxf = jnp.clip(idx_f.astype(jnp.int32), 0, K - 2)
            w_ce = idx_f - idxf.astype(jnp.float32)
            kf = jnp.sum(jnp.where(iota_k == idxf, kmh, 0.0), axis=-1, keepdims=True)
            kc = jnp.sum(jnp.where(iota_k == idxf + 1, kmh, 0.0), axis=-1, keepdims=True)
            ker = (kf * (1.0 - w_ce) + kc * w_ce) * pwr * valid
            wf_cols.append(ker * (1.0 - frac))
            wc_cols.append(ker * frac)
            sfi_cols.append((sfloor - (l0 - HALO)) * D)
        for _ in range(SP - S):
            wf_cols.append(jnp.zeros((T, 1), jnp.float32))
            wc_cols.append(jnp.zeros((T, 1), jnp.float32))
            sfi_cols.append(jnp.zeros((T, 1), jnp.int32))
    wf_ref[0] = jnp.concatenate(wf_cols, axis=1)
    wc_ref[0] = jnp.concatenate(wc_cols, axis=1)
    sfi_ref[0] = jnp.concatenate(sfi_cols, axis=1)


def _stage_a(x, Ww, bw, gw, Wk, bk, gk, We, be, ge):
    full = lambda shape: pl.BlockSpec(shape, lambda b, i: (0,) * len(shape))
    return pl.pallas_call(
        _a_body,
        grid=(B, NBLK),
        in_specs=[
            pl.BlockSpec((1, T, C), lambda b, i: (b, i, 0)),
            full((C, 128)), full((128,)), full((128,)),
            full((C, H * K)), full((H * K,)), full((H * K,)),
            full((C, 128)), full((128,)), full((128,)),
        ],
        out_specs=[
            pl.BlockSpec((1, T, HS), lambda b, i: (b, i, 0)),
            pl.BlockSpec((1, T, HS), lambda b, i: (b, i, 0)),
            pl.BlockSpec((1, T, HS), lambda b, i: (b, i, 0)),
        ],
        out_shape=[
            jax.ShapeDtypeStruct((B, L, HS), jnp.float32),
            jax.ShapeDtypeStruct((B, L, HS), jnp.float32),
            jax.ShapeDtypeStruct((B, L, HS), jnp.int32),
        ],
    )(x, Ww, bw, gw, Wk, bk, gk, We, be, ge)


def _b_body(xh_ref, wf_ref, wc_ref, sfi_ref, out_ref,
            xwin, wfv, wcv, sfiv, outv):
    wid = lax.axis_index("s") * NC + lax.axis_index("c")

    def task_body(t, carry):
        tid = wid * (TASKS // NW) + t
        b = tid // (H * NBLK)
        r = tid % (H * NBLK)
        h = r // NBLK
        blk = r % NBLK
        pltpu.sync_copy(xh_ref.at[b, h, pl.ds(blk * T * D, WC * D)], xwin)
        pltpu.sync_copy(wf_ref.at[b, pl.ds(h * SP, SP), pl.ds(blk * T, T)], wfv)
        pltpu.sync_copy(wc_ref.at[b, pl.ds(h * SP, SP), pl.ds(blk * T, T)], wcv)
        pltpu.sync_copy(sfi_ref.at[b, pl.ds(h * SP, SP), pl.ds(blk * T, T)], sfiv)

        def l_body(g, carry2):
            base = g * 16
            wfs = [wfv[j, pl.ds(base, 16)] for j in range(S)]
            wcs = [wcv[j, pl.ds(base, 16)] for j in range(S)]
            idxf = [sfiv[j, pl.ds(base, 16)] for j in range(S)]
            idxc = [ix + D for ix in idxf]

            @plsc.parallel_loop(0, D, 1, unroll=4)
            def d_body(d):
                vals = []
                for j in range(S):
                    vf = plsc.load_gather(xwin, [idxf[j] + d])
                    vc = plsc.load_gather(xwin, [idxc[j] + d])
                    vals.append(wfs[j] * vf + wcs[j] * vc)
                while len(vals) > 1:
                    vals = [a + b for a, b in zip(vals[::2], vals[1::2])] + \
                        ([vals[-1]] if len(vals) % 2 else [])
                outv[pl.ds(d * T + base, 16)] = vals[0]

            return carry2

        lax.fori_loop(0, T // 16, l_body, 0)
        pltpu.sync_copy(outv, out_ref.at[b, h, blk])
        return carry

    lax.fori_loop(0, TASKS // NW, task_body, 0)


def _stage_b(xh, wf_t, wc_t, sfi_t):
    mesh = plsc.VectorSubcoreMesh(core_axis_name="c", subcore_axis_name="s",
                                  num_cores=NC, num_subcores=NW // NC)
    return pl.kernel(
        _b_body,
        out_type=jax.ShapeDtypeStruct((B, H, NBLK, D * T), jnp.float32),
        mesh=mesh,
        compiler_params=pltpu.CompilerParams(needs_layout_passes=False),
        scratch_types=[
            pltpu.VMEM((WC * D,), jnp.float32),
            pltpu.VMEM((SP, T), jnp.float32),
            pltpu.VMEM((SP, T), jnp.float32),
            pltpu.VMEM((SP, T), jnp.int32),
            pltpu.VMEM((D * T,), jnp.float32),
        ],
    )(xh, wf_t, wc_t, sfi_t)


def _c_body(xt_ref, wo_ref, o_ref):
    h = pl.program_id(2)

    @pl.when(h == 0)
    def _():
        o_ref[...] = jnp.zeros_like(o_ref)

    o_ref[0] += lax.dot_general(
        xt_ref[0, 0, 0], wo_ref[...],
        (((0,), (0,)), ((), ())),
        preferred_element_type=jnp.float32)


def _stage_c(acc_t, WoT):
    return pl.pallas_call(
        _c_body,
        grid=(B, NBLK, H),
        in_specs=[
            pl.BlockSpec((1, 1, 1, D, T), lambda b, i, h: (b, h, i, 0, 0)),
            pl.BlockSpec((D, C), lambda b, i, h: (h, 0)),
        ],
        out_specs=pl.BlockSpec((1, T, C), lambda b, i, h: (b, i, 0)),
        out_shape=jax.ShapeDtypeStruct((B, L, C), jnp.float32),
    )(acc_t, WoT)


@jax.jit
def kernel(x, W_wave, b_wave, wave_gamma, W_kernel, b_kernel, kernel_gamma,
           W_exp, b_exp, exp_gamma, W_out):
    f32 = jnp.float32
    Ww = jnp.zeros((C, 128), f32).at[:, :2 * H].set(W_wave.T)
    bw = jnp.zeros((128,), f32).at[:2 * H].set(b_wave)
    gw = jnp.zeros((128,), f32).at[:2 * H].set(wave_gamma)
    We = jnp.zeros((C, 128), f32).at[:, :1].set(W_exp.T)
    be = jnp.zeros((128,), f32).at[:1].set(b_exp)
    ge = jnp.zeros((128,), f32).at[:1].set(exp_gamma)

    wf, wc, sfi = _stage_a(x, Ww, bw, gw, W_kernel.T, b_kernel, kernel_gamma,
                           We, be, ge)
    wf_t = wf.transpose(0, 2, 1)
    wc_t = wc.transpose(0, 2, 1)
    sfi_t = sfi.transpose(0, 2, 1)

    x_pad = jnp.pad(x, ((0, 0), (HALO, HALO), (0, 0)))
    xh = (x_pad.reshape(B, L + 2 * HALO, H, D)
          .transpose(0, 2, 1, 3)
          .reshape(B, H, (L + 2 * HALO) * D))

    acc_t = _stage_b(xh, wf_t, wc_t, sfi_t)
    return _stage_c(acc_t.reshape(B, H, NBLK, D, T), W_out.T)
